# Initial kernel scaffold; baseline (speedup 1.0000x reference)
import jax
import jax.numpy as jnp
from jax.experimental import pallas as pl


def kernel(y_pred, y_true):
    raise NotImplementedError("write your pallas kernel here")


# speedup vs baseline: 7.8910x; 7.8910x over previous
import jax
import jax.numpy as jnp
from jax.experimental import pallas as pl
from jax.experimental.pallas import tpu as pltpu

_NNK = 16
_ROWS = 512


def _body(yt_rows, yp_cols, yt_cols, out, mincol_acc):
    b = pl.program_id(0)
    i = pl.program_id(1)
    ni = pl.num_programs(1)
    n = yp_cols.shape[2]

    @pl.when((b == 0) & (i == 0))
    def _init():
        out[0] = 0.0
        out[1] = 0.0
        out[2] = 0.0

    xt = yt_rows[0]
    yp = yp_cols[0]
    yt = yt_cols[0]

    x0 = xt[:, 0:1]
    x1 = xt[:, 1:2]
    x2 = xt[:, 2:3]
    da = (x0 - yp[0:1, :]) ** 2 + (x1 - yp[1:2, :]) ** 2 + (x2 - yp[2:3, :]) ** 2
    db = (x0 - yt[0:1, :]) ** 2 + (x1 - yt[1:2, :]) ** 2 + (x2 - yt[2:3, :]) ** 2

    colmin = jnp.min(da, axis=0, keepdims=True)

    @pl.when(i == 0)
    def _cm0():
        mincol_acc[...] = colmin

    @pl.when(i != 0)
    def _cm1():
        mincol_acc[...] = jnp.minimum(mincol_acc[...], colmin)

    iota = jax.lax.broadcasted_iota(jnp.int32, (_ROWS, n), 1)
    inf = jnp.float32(jnp.inf)
    big = jnp.int32(2**30)

    def extract(v):
        m = jnp.min(v, axis=1, keepdims=True)
        idx = jnp.min(jnp.where(v == m, iota, big), axis=1, keepdims=True)
        v = jnp.where(iota == idx, inf, v)
        return v, m

    acc_abs = jnp.zeros((_ROWS, 1), jnp.float32)
    va, vb = da, db
    minrow = None
    for k in range(_NNK):
        va, ma = extract(va)
        vb, mb = extract(vb)
        if k == 0:
            minrow = ma
        acc_abs = acc_abs + jnp.abs(jnp.sqrt(ma) - jnp.sqrt(mb))

    out[0] += jnp.sum(jnp.sqrt(minrow))
    out[2] += jnp.sum(acc_abs)

    @pl.when(i == ni - 1)
    def _fin():
        out[1] += jnp.sum(jnp.sqrt(mincol_acc[...]))


@jax.jit
def kernel(y_pred, y_true):
    bsz, n, _ = y_pred.shape
    yp_cols = jnp.transpose(y_pred, (0, 2, 1))
    yt_cols = jnp.transpose(y_true, (0, 2, 1))
    sums = pl.pallas_call(
        _body,
        grid=(bsz, n // _ROWS),
        in_specs=[
            pl.BlockSpec((1, _ROWS, 3), lambda b, i: (b, i, 0)),
            pl.BlockSpec((1, 3, n), lambda b, i: (b, 0, 0)),
            pl.BlockSpec((1, 3, n), lambda b, i: (b, 0, 0)),
        ],
        out_specs=pl.BlockSpec(memory_space=pltpu.SMEM),
        out_shape=jax.ShapeDtypeStruct((3,), jnp.float32),
        scratch_shapes=[pltpu.VMEM((1, n), jnp.float32)],
    )(y_true, yp_cols, yt_cols)
    n_rows = bsz * n
    shape_loss = (sums[0] / n_rows + sums[1] / n_rows) * 0.5
    density_loss = sums[2] / (n_rows * _NNK)
    data_loss = shape_loss + density_loss
    return (data_loss, shape_loss, density_loss)


# speedup vs baseline: 8.7627x; 1.1105x over previous
import jax
import jax.numpy as jnp
from jax.experimental import pallas as pl
from jax.experimental.pallas import tpu as pltpu

_NNK = 16
_ROWS = 1024


def _body(yt_rows, yp_cols, yt_cols, out, mincol_acc):
    b = pl.program_id(0)
    i = pl.program_id(1)
    ni = pl.num_programs(1)
    n = yp_cols.shape[2]

    @pl.when((b == 0) & (i == 0))
    def _init():
        out[0] = 0.0
        out[1] = 0.0
        out[2] = 0.0

    xt = yt_rows[0]
    yp = yp_cols[0]
    yt = yt_cols[0]

    x0 = xt[:, 0:1]
    x1 = xt[:, 1:2]
    x2 = xt[:, 2:3]
    da = (x0 - yp[0:1, :]) ** 2 + (x1 - yp[1:2, :]) ** 2 + (x2 - yp[2:3, :]) ** 2
    db = (x0 - yt[0:1, :]) ** 2 + (x1 - yt[1:2, :]) ** 2 + (x2 - yt[2:3, :]) ** 2

    colmin = jnp.min(da, axis=0, keepdims=True)

    @pl.when(i == 0)
    def _cm0():
        mincol_acc[...] = colmin

    @pl.when(i != 0)
    def _cm1():
        mincol_acc[...] = jnp.minimum(mincol_acc[...], colmin)

    iota = jax.lax.broadcasted_iota(jnp.int32, (_ROWS, n), 1)
    inf = jnp.float32(jnp.inf)
    big = jnp.int32(2**30)

    def extract(v):
        m = jnp.min(v, axis=1, keepdims=True)
        idx = jnp.min(jnp.where(v == m, iota, big), axis=1, keepdims=True)
        v = jnp.where(iota == idx, inf, v)
        return v, m

    acc_abs = jnp.zeros((_ROWS, 1), jnp.float32)
    va, vb = da, db
    minrow = None
    for k in range(_NNK):
        va, ma = extract(va)
        vb, mb = extract(vb)
        if k == 0:
            minrow = ma
        acc_abs = acc_abs + jnp.abs(jnp.sqrt(ma) - jnp.sqrt(mb))

    out[0] += jnp.sum(jnp.sqrt(minrow))
    out[2] += jnp.sum(acc_abs)

    @pl.when(i == ni - 1)
    def _fin():
        out[1] += jnp.sum(jnp.sqrt(mincol_acc[...]))


@jax.jit
def kernel(y_pred, y_true):
    bsz, n, _ = y_pred.shape
    yp_cols = jnp.transpose(y_pred, (0, 2, 1))
    yt_cols = jnp.transpose(y_true, (0, 2, 1))
    sums = pl.pallas_call(
        _body,
        grid=(bsz, n // _ROWS),
        in_specs=[
            pl.BlockSpec((1, _ROWS, 3), lambda b, i: (b, i, 0)),
            pl.BlockSpec((1, 3, n), lambda b, i: (b, 0, 0)),
            pl.BlockSpec((1, 3, n), lambda b, i: (b, 0, 0)),
        ],
        out_specs=pl.BlockSpec(memory_space=pltpu.SMEM),
        out_shape=jax.ShapeDtypeStruct((3,), jnp.float32),
        scratch_shapes=[pltpu.VMEM((1, n), jnp.float32)],
    )(y_true, yp_cols, yt_cols)
    n_rows = bsz * n
    shape_loss = (sums[0] / n_rows + sums[1] / n_rows) * 0.5
    density_loss = sums[2] / (n_rows * _NNK)
    data_loss = shape_loss + density_loss
    return (data_loss, shape_loss, density_loss)


# speedup vs baseline: 8.9724x; 1.0239x over previous
import jax
import jax.numpy as jnp
from jax.experimental import pallas as pl
from jax.experimental.pallas import tpu as pltpu

_NNK = 16
_ROWS = 2048


def _body(yt_rows, yp_cols, yt_cols, out, mincol_acc):
    b = pl.program_id(0)
    i = pl.program_id(1)
    ni = pl.num_programs(1)
    n = yp_cols.shape[2]

    @pl.when((b == 0) & (i == 0))
    def _init():
        out[0] = 0.0
        out[1] = 0.0
        out[2] = 0.0

    xt = yt_rows[0]
    yp = yp_cols[0]
    yt = yt_cols[0]

    x0 = xt[:, 0:1]
    x1 = xt[:, 1:2]
    x2 = xt[:, 2:3]
    da = (x0 - yp[0:1, :]) ** 2 + (x1 - yp[1:2, :]) ** 2 + (x2 - yp[2:3, :]) ** 2
    db = (x0 - yt[0:1, :]) ** 2 + (x1 - yt[1:2, :]) ** 2 + (x2 - yt[2:3, :]) ** 2

    colmin = jnp.min(da, axis=0, keepdims=True)

    @pl.when(i == 0)
    def _cm0():
        mincol_acc[...] = colmin

    @pl.when(i != 0)
    def _cm1():
        mincol_acc[...] = jnp.minimum(mincol_acc[...], colmin)

    iota = jax.lax.broadcasted_iota(jnp.int32, (_ROWS, n), 1)
    inf = jnp.float32(jnp.inf)
    big = jnp.int32(2**30)

    def extract(v):
        m = jnp.min(v, axis=1, keepdims=True)
        idx = jnp.min(jnp.where(v == m, iota, big), axis=1, keepdims=True)
        v = jnp.where(iota == idx, inf, v)
        return v, m

    acc_abs = jnp.zeros((_ROWS, 1), jnp.float32)
    va, vb = da, db
    minrow = None
    for k in range(_NNK):
        va, ma = extract(va)
        vb, mb = extract(vb)
        if k == 0:
            minrow = ma
        acc_abs = acc_abs + jnp.abs(jnp.sqrt(ma) - jnp.sqrt(mb))

    out[0] += jnp.sum(jnp.sqrt(minrow))
    out[2] += jnp.sum(acc_abs)

    @pl.when(i == ni - 1)
    def _fin():
        out[1] += jnp.sum(jnp.sqrt(mincol_acc[...]))


@jax.jit
def kernel(y_pred, y_true):
    bsz, n, _ = y_pred.shape
    yp_cols = jnp.transpose(y_pred, (0, 2, 1))
    yt_cols = jnp.transpose(y_true, (0, 2, 1))
    sums = pl.pallas_call(
        _body,
        grid=(bsz, n // _ROWS),
        in_specs=[
            pl.BlockSpec((1, _ROWS, 3), lambda b, i: (b, i, 0)),
            pl.BlockSpec((1, 3, n), lambda b, i: (b, 0, 0)),
            pl.BlockSpec((1, 3, n), lambda b, i: (b, 0, 0)),
        ],
        out_specs=pl.BlockSpec(memory_space=pltpu.SMEM),
        out_shape=jax.ShapeDtypeStruct((3,), jnp.float32),
        scratch_shapes=[pltpu.VMEM((1, n), jnp.float32)],
    )(y_true, yp_cols, yt_cols)
    n_rows = bsz * n
    shape_loss = (sums[0] / n_rows + sums[1] / n_rows) * 0.5
    density_loss = sums[2] / (n_rows * _NNK)
    data_loss = shape_loss + density_loss
    return (data_loss, shape_loss, density_loss)


# speedup vs baseline: 9.9550x; 1.1095x over previous
import jax
import jax.numpy as jnp
from jax.experimental import pallas as pl
from jax.experimental.pallas import tpu as pltpu

_NNK = 16
_ROWS = 1024


def _body(yt_rows, yp_cols, yt_cols, out, mincol_acc):
    b = pl.program_id(0)
    i = pl.program_id(1)
    ni = pl.num_programs(1)
    n = yp_cols.shape[2]

    @pl.when((b == 0) & (i == 0))
    def _init():
        out[0] = 0.0
        out[1] = 0.0
        out[2] = 0.0

    xt = yt_rows[0]
    yp = yp_cols[0]
    yt = yt_cols[0]

    x0 = xt[:, 0:1]
    x1 = xt[:, 1:2]
    x2 = xt[:, 2:3]
    da = (x0 - yp[0:1, :]) ** 2 + (x1 - yp[1:2, :]) ** 2 + (x2 - yp[2:3, :]) ** 2
    db = (x0 - yt[0:1, :]) ** 2 + (x1 - yt[1:2, :]) ** 2 + (x2 - yt[2:3, :]) ** 2

    colmin = jnp.min(da, axis=0, keepdims=True)

    @pl.when(i == 0)
    def _cm0():
        mincol_acc[...] = colmin

    @pl.when(i != 0)
    def _cm1():
        mincol_acc[...] = jnp.minimum(mincol_acc[...], colmin)

    iota = jax.lax.broadcasted_iota(jnp.int32, (_ROWS, n), 1).astype(jnp.float32)
    inf = jnp.float32(jnp.inf)
    big = jnp.float32(3e38)

    def extract(v):
        m = jnp.min(v, axis=1, keepdims=True)
        t = jnp.where(v == m, iota, big)
        idx = jnp.min(t, axis=1, keepdims=True)
        v = jnp.where(t == idx, inf, v)
        return v, m

    acc_abs = jnp.zeros((_ROWS, 1), jnp.float32)
    va, vb = da, db
    minrow = None
    for k in range(_NNK):
        va, ma = extract(va)
        vb, mb = extract(vb)
        if k == 0:
            minrow = ma
        acc_abs = acc_abs + jnp.abs(jnp.sqrt(ma) - jnp.sqrt(mb))

    out[0] += jnp.sum(jnp.sqrt(minrow))
    out[2] += jnp.sum(acc_abs)

    @pl.when(i == ni - 1)
    def _fin():
        out[1] += jnp.sum(jnp.sqrt(mincol_acc[...]))


@jax.jit
def kernel(y_pred, y_true):
    bsz, n, _ = y_pred.shape
    yp_cols = jnp.transpose(y_pred, (0, 2, 1))
    yt_cols = jnp.transpose(y_true, (0, 2, 1))
    sums = pl.pallas_call(
        _body,
        grid=(bsz, n // _ROWS),
        in_specs=[
            pl.BlockSpec((1, _ROWS, 3), lambda b, i: (b, i, 0)),
            pl.BlockSpec((1, 3, n), lambda b, i: (b, 0, 0)),
            pl.BlockSpec((1, 3, n), lambda b, i: (b, 0, 0)),
        ],
        out_specs=pl.BlockSpec(memory_space=pltpu.SMEM),
        out_shape=jax.ShapeDtypeStruct((3,), jnp.float32),
        scratch_shapes=[pltpu.VMEM((1, n), jnp.float32)],
    )(y_true, yp_cols, yt_cols)
    n_rows = bsz * n
    shape_loss = (sums[0] / n_rows + sums[1] / n_rows) * 0.5
    density_loss = sums[2] / (n_rows * _NNK)
    data_loss = shape_loss + density_loss
    return (data_loss, shape_loss, density_loss)


# speedup vs baseline: 10.1488x; 1.0195x over previous
import jax
import jax.numpy as jnp
from jax.experimental import pallas as pl
from jax.experimental.pallas import tpu as pltpu

_NNK = 16
_ROWS = 1024


def _body(yt_rows, yp_cols, yt_cols, out, mincol_acc):
    b = pl.program_id(0)
    i = pl.program_id(1)
    ni = pl.num_programs(1)
    n = yp_cols.shape[2]

    @pl.when((b == 0) & (i == 0))
    def _init():
        out[0] = 0.0
        out[1] = 0.0
        out[2] = 0.0

    xt = yt_rows[0]
    yp = yp_cols[0]
    yt = yt_cols[0]

    x0 = xt[:, 0:1]
    x1 = xt[:, 1:2]
    x2 = xt[:, 2:3]
    da = (x0 - yp[0:1, :]) ** 2 + (x1 - yp[1:2, :]) ** 2 + (x2 - yp[2:3, :]) ** 2
    db = (x0 - yt[0:1, :]) ** 2 + (x1 - yt[1:2, :]) ** 2 + (x2 - yt[2:3, :]) ** 2

    colmin = jnp.min(da, axis=0, keepdims=True)

    @pl.when(i == 0)
    def _cm0():
        mincol_acc[...] = colmin

    @pl.when(i != 0)
    def _cm1():
        mincol_acc[...] = jnp.minimum(mincol_acc[...], colmin)

    iota = jax.lax.broadcasted_iota(jnp.int32, (_ROWS, n), 1).astype(jnp.float32)
    inf = jnp.float32(jnp.inf)
    big = jnp.float32(3e38)

    def extract(v):
        m = jnp.min(v, axis=1, keepdims=True)
        t = jnp.where(v == m, iota, big)
        idx = jnp.min(t, axis=1, keepdims=True)
        v = jnp.where(t == idx, inf, v)
        return v, m

    row_iota = jax.lax.broadcasted_iota(jnp.int32, (_ROWS, n), 0)
    col_iota = jax.lax.broadcasted_iota(jnp.int32, (_ROWS, n), 1)
    db = jnp.where(col_iota == row_iota + i * _ROWS, inf, db)

    va, minrow = extract(da)
    acc_abs = jnp.sqrt(minrow)
    vb = db
    for k in range(1, _NNK):
        va, ma = extract(va)
        vb, mb = extract(vb)
        acc_abs = acc_abs + jnp.abs(jnp.sqrt(ma) - jnp.sqrt(mb))

    out[0] += jnp.sum(jnp.sqrt(minrow))
    out[2] += jnp.sum(acc_abs)

    @pl.when(i == ni - 1)
    def _fin():
        out[1] += jnp.sum(jnp.sqrt(mincol_acc[...]))


@jax.jit
def kernel(y_pred, y_true):
    bsz, n, _ = y_pred.shape
    yp_cols = jnp.transpose(y_pred, (0, 2, 1))
    yt_cols = jnp.transpose(y_true, (0, 2, 1))
    sums = pl.pallas_call(
        _body,
        grid=(bsz, n // _ROWS),
        in_specs=[
            pl.BlockSpec((1, _ROWS, 3), lambda b, i: (b, i, 0)),
            pl.BlockSpec((1, 3, n), lambda b, i: (b, 0, 0)),
            pl.BlockSpec((1, 3, n), lambda b, i: (b, 0, 0)),
        ],
        out_specs=pl.BlockSpec(memory_space=pltpu.SMEM),
        out_shape=jax.ShapeDtypeStruct((3,), jnp.float32),
        scratch_shapes=[pltpu.VMEM((1, n), jnp.float32)],
    )(y_true, yp_cols, yt_cols)
    n_rows = bsz * n
    shape_loss = (sums[0] / n_rows + sums[1] / n_rows) * 0.5
    density_loss = sums[2] / (n_rows * _NNK)
    data_loss = shape_loss + density_loss
    return (data_loss, shape_loss, density_loss)
